# Initial kernel scaffold; baseline (speedup 1.0000x reference)
import jax
import jax.numpy as jnp
from jax.experimental import pallas as pl


def kernel(x, y, Wq, Wkv, ln_g, ln_b, Wp, bp, a1, a2):
    raise NotImplementedError("write your pallas kernel here")


# speedup vs baseline: 71.2579x; 71.2579x over previous
import jax
import jax.numpy as jnp
import numpy as np
from jax.experimental import pallas as pl

_HEADS = 8
_INT_MIN = np.int32(-2147483648)


def _avg_pool(x, k, p):
    s = jax.lax.reduce_window(x, 0.0, jax.lax.add, (1, 1, k, k), (1, 1, 1, 1),
                              ((0, 0), (0, 0), (p, p), (p, p)))
    return s / float(k * k)


def _kth_threshold_keys(keys, kk):
    n = keys.shape[0]
    lo = jnp.zeros((n, 1), jnp.int32)

    def body(i, lo):
        bit = jnp.left_shift(jnp.int32(1), 31 - i)
        cand = jnp.bitwise_or(lo, bit)
        thr_s = jnp.bitwise_xor(cand, _INT_MIN)
        cnt = jnp.sum((keys >= thr_s).astype(jnp.int32), axis=1, keepdims=True)
        return jnp.where(cnt >= kk, cand, lo)

    lo = jax.lax.fori_loop(0, 32, body, lo)
    return jnp.bitwise_xor(lo, _INT_MIN)


def _attn_body(scale, k1, k2, q_ref, k_ref, v_ref, wp_ref, a1_ref, a2_ref,
               bp_ref, out_ref):
    h = pl.program_id(1)
    q = q_ref[0, 0]
    k = k_ref[0, 0]
    attn = jax.lax.dot_general(
        q, k, (((1,), (1,)), ((), ())),
        preferred_element_type=jnp.float32) * scale

    bits = jax.lax.bitcast_convert_type(attn, jnp.int32)
    keys = jnp.where(bits >= 0, bits, jnp.bitwise_xor(bits, jnp.int32(0x7FFFFFFF)))
    t1 = _kth_threshold_keys(keys, k1)
    t2 = _kth_threshold_keys(keys, k2)

    rowmax = jnp.max(attn, axis=1, keepdims=True)
    e = jnp.exp(attn - rowmax)
    m1 = (keys >= t1).astype(jnp.float32)
    m2 = (keys >= t2).astype(jnp.float32)
    s1 = jnp.sum(e * m1, axis=1, keepdims=True)
    s2 = jnp.sum(e * m2, axis=1, keepdims=True)
    wgt = e * (m1 * (a1_ref[...] / s1) + m2 * (a2_ref[...] / s2))

    out_h = jax.lax.dot_general(
        wgt, v_ref[0, 0], (((1,), (0,)), ((), ())),
        preferred_element_type=jnp.float32,
        precision=jax.lax.Precision.HIGHEST)
    contrib = jnp.dot(out_h, wp_ref[0], preferred_element_type=jnp.float32,
                      precision=jax.lax.Precision.HIGHEST)

    @pl.when(h == 0)
    def _():
        out_ref[0] = contrib + bp_ref[...]

    @pl.when(h != 0)
    def _():
        out_ref[0] = out_ref[0] + contrib


def kernel(x, y, Wq, Wkv, ln_g, ln_b, Wp, bp, a1, a2):
    num_heads = _HEADS
    B, C, H, W = x.shape
    hd = C // num_heads
    scale = hd ** (-0.5)
    N = H * W

    yy = _avg_pool(y, 3, 1) + _avg_pool(y, 5, 2) + _avg_pool(y, 7, 3)
    yy = yy.reshape(B, C, N).transpose(0, 2, 1)
    mu = yy.mean(-1, keepdims=True)
    var = ((yy - mu) ** 2).mean(-1, keepdims=True)
    yy = (yy - mu) / jnp.sqrt(var + 1e-5) * ln_g + ln_b
    xt = x.reshape(B, C, N).transpose(0, 2, 1)
    kv = (yy @ Wkv).reshape(B, N, 2, num_heads, hd).transpose(2, 0, 3, 1, 4)
    k_, v_ = kv[0], kv[1]
    q = (xt @ Wq).reshape(B, N, num_heads, hd).transpose(0, 2, 1, 3)

    wp3 = Wp.reshape(num_heads, hd, C)
    a1b = a1.reshape(1, 1)
    a2b = a2.reshape(1, 1)
    bp2 = bp.reshape(1, C)
    kk1, kk2 = N // 2, N // 3

    import functools
    body = functools.partial(_attn_body, np.float32(scale), kk1, kk2)

    xo = pl.pallas_call(
        body,
        grid=(B, num_heads),
        in_specs=[
            pl.BlockSpec((1, 1, N, hd), lambda b, h: (b, h, 0, 0)),
            pl.BlockSpec((1, 1, N, hd), lambda b, h: (b, h, 0, 0)),
            pl.BlockSpec((1, 1, N, hd), lambda b, h: (b, h, 0, 0)),
            pl.BlockSpec((1, hd, C), lambda b, h: (h, 0, 0)),
            pl.BlockSpec((1, 1), lambda b, h: (0, 0)),
            pl.BlockSpec((1, 1), lambda b, h: (0, 0)),
            pl.BlockSpec((1, C), lambda b, h: (0, 0)),
        ],
        out_specs=pl.BlockSpec((1, N, C), lambda b, h: (b, 0, 0)),
        out_shape=jax.ShapeDtypeStruct((B, N, C), jnp.float32),
    )(q, k_, v_, wp3, a1b, a2b, bp2)

    return xo.reshape(B, H, W, C).transpose(0, 3, 1, 2)


# speedup vs baseline: 72.8423x; 1.0222x over previous
import jax
import jax.numpy as jnp
import numpy as np
from jax.experimental import pallas as pl

_HEADS = 8
_INT_MIN = np.int32(-2147483648)


def _avg_pool(x, k, p):
    s = jax.lax.reduce_window(x, 0.0, jax.lax.add, (1, 1, k, k), (1, 1, 1, 1),
                              ((0, 0), (0, 0), (p, p), (p, p)))
    return s / float(k * k)


def _kth_threshold_keys2(keys, kk1, kk2):
    n = keys.shape[0]
    z = jnp.zeros((n, 1), jnp.int32)

    def body(i, c):
        lo1, lo2 = c
        bit = jnp.left_shift(jnp.int32(1), 31 - i)
        cand1 = jnp.bitwise_or(lo1, bit)
        cand2 = jnp.bitwise_or(lo2, bit)
        thr1 = jnp.bitwise_xor(cand1, _INT_MIN)
        thr2 = jnp.bitwise_xor(cand2, _INT_MIN)
        cnt1 = jnp.sum((keys >= thr1).astype(jnp.int32), axis=1, keepdims=True)
        cnt2 = jnp.sum((keys >= thr2).astype(jnp.int32), axis=1, keepdims=True)
        return (jnp.where(cnt1 >= kk1, cand1, lo1),
                jnp.where(cnt2 >= kk2, cand2, lo2))

    lo1, lo2 = jax.lax.fori_loop(0, 32, body, (z, z))
    return (jnp.bitwise_xor(lo1, _INT_MIN),
            jnp.bitwise_xor(lo2, _INT_MIN))


def _attn_body(scale, k1, k2, q_ref, k_ref, v_ref, wp_ref, a1_ref, a2_ref,
               bp_ref, out_ref):
    h = pl.program_id(1)
    q = q_ref[0, 0]
    k = k_ref[0, 0]
    attn = jax.lax.dot_general(
        q, k, (((1,), (1,)), ((), ())),
        preferred_element_type=jnp.float32) * scale

    bits = jax.lax.bitcast_convert_type(attn, jnp.int32)
    keys = jnp.where(bits >= 0, bits, jnp.bitwise_xor(bits, jnp.int32(0x7FFFFFFF)))
    t1, t2 = _kth_threshold_keys2(keys, k1, k2)

    rowmax = jnp.max(attn, axis=1, keepdims=True)
    e = jnp.exp(attn - rowmax)
    m1 = (keys >= t1).astype(jnp.float32)
    m2 = (keys >= t2).astype(jnp.float32)
    s1 = jnp.sum(e * m1, axis=1, keepdims=True)
    s2 = jnp.sum(e * m2, axis=1, keepdims=True)
    wgt = e * (m1 * (a1_ref[...] / s1) + m2 * (a2_ref[...] / s2))

    out_h = jax.lax.dot_general(
        wgt, v_ref[0, 0], (((1,), (0,)), ((), ())),
        preferred_element_type=jnp.float32,
        precision=jax.lax.Precision.HIGHEST)
    contrib = jnp.dot(out_h, wp_ref[0], preferred_element_type=jnp.float32,
                      precision=jax.lax.Precision.HIGHEST)

    @pl.when(h == 0)
    def _():
        out_ref[0] = contrib + bp_ref[...]

    @pl.when(h != 0)
    def _():
        out_ref[0] = out_ref[0] + contrib


def kernel(x, y, Wq, Wkv, ln_g, ln_b, Wp, bp, a1, a2):
    num_heads = _HEADS
    B, C, H, W = x.shape
    hd = C // num_heads
    scale = hd ** (-0.5)
    N = H * W

    yy = _avg_pool(y, 3, 1) + _avg_pool(y, 5, 2) + _avg_pool(y, 7, 3)
    yy = yy.reshape(B, C, N).transpose(0, 2, 1)
    mu = yy.mean(-1, keepdims=True)
    var = ((yy - mu) ** 2).mean(-1, keepdims=True)
    yy = (yy - mu) / jnp.sqrt(var + 1e-5) * ln_g + ln_b
    xt = x.reshape(B, C, N).transpose(0, 2, 1)
    kv = (yy @ Wkv).reshape(B, N, 2, num_heads, hd).transpose(2, 0, 3, 1, 4)
    k_, v_ = kv[0], kv[1]
    q = (xt @ Wq).reshape(B, N, num_heads, hd).transpose(0, 2, 1, 3)

    wp3 = Wp.reshape(num_heads, hd, C)
    a1b = a1.reshape(1, 1)
    a2b = a2.reshape(1, 1)
    bp2 = bp.reshape(1, C)
    kk1, kk2 = N // 2, N // 3

    import functools
    body = functools.partial(_attn_body, np.float32(scale), kk1, kk2)

    xo = pl.pallas_call(
        body,
        grid=(B, num_heads),
        in_specs=[
            pl.BlockSpec((1, 1, N, hd), lambda b, h: (b, h, 0, 0)),
            pl.BlockSpec((1, 1, N, hd), lambda b, h: (b, h, 0, 0)),
            pl.BlockSpec((1, 1, N, hd), lambda b, h: (b, h, 0, 0)),
            pl.BlockSpec((1, hd, C), lambda b, h: (h, 0, 0)),
            pl.BlockSpec((1, 1), lambda b, h: (0, 0)),
            pl.BlockSpec((1, 1), lambda b, h: (0, 0)),
            pl.BlockSpec((1, C), lambda b, h: (0, 0)),
        ],
        out_specs=pl.BlockSpec((1, N, C), lambda b, h: (b, 0, 0)),
        out_shape=jax.ShapeDtypeStruct((B, N, C), jnp.float32),
    )(q, k_, v_, wp3, a1b, a2b, bp2)

    return xo.reshape(B, H, W, C).transpose(0, 3, 1, 2)


# speedup vs baseline: 76.9272x; 1.0561x over previous
import jax
import jax.numpy as jnp
import numpy as np
from jax.experimental import pallas as pl

_HEADS = 8
_INT_MIN = np.int32(-2147483648)


def _avg_pool(x, k, p):
    s = jax.lax.reduce_window(x, 0.0, jax.lax.add, (1, 1, k, k), (1, 1, 1, 1),
                              ((0, 0), (0, 0), (p, p), (p, p)))
    return s / float(k * k)


def _kth_threshold_keys2(keys, kk1, kk2):
    n = keys.shape[0]
    z = jnp.zeros((n, 1), jnp.int32)

    def body(i, c):
        lo1, lo2 = c
        bit = jnp.left_shift(jnp.int32(1), 31 - i)
        cand1 = jnp.bitwise_or(lo1, bit)
        cand2 = jnp.bitwise_or(lo2, bit)
        thr1 = jnp.bitwise_xor(cand1, _INT_MIN)
        thr2 = jnp.bitwise_xor(cand2, _INT_MIN)
        comb = ((keys >= thr1).astype(jnp.int32)
                + jnp.left_shift((keys >= thr2).astype(jnp.int32), 11))
        cntp = jnp.sum(comb, axis=1, keepdims=True)
        cnt1 = jnp.bitwise_and(cntp, 2047)
        cnt2 = jnp.right_shift(cntp, 11)
        return (jnp.where(cnt1 >= kk1, cand1, lo1),
                jnp.where(cnt2 >= kk2, cand2, lo2))

    lo1, lo2 = jax.lax.fori_loop(0, 32, body, (z, z))
    return (jnp.bitwise_xor(lo1, _INT_MIN),
            jnp.bitwise_xor(lo2, _INT_MIN))


def _attn_body(scale, k1, k2, q_ref, k_ref, v_ref, wp_ref, a1_ref, a2_ref,
               bp_ref, out_ref):
    h = pl.program_id(1)
    q = q_ref[0, 0]
    k = k_ref[0, 0]
    attn = jax.lax.dot_general(
        q, k, (((1,), (1,)), ((), ())),
        preferred_element_type=jnp.float32) * scale

    bits = jax.lax.bitcast_convert_type(attn, jnp.int32)
    keys = jnp.where(bits >= 0, bits, jnp.bitwise_xor(bits, jnp.int32(0x7FFFFFFF)))
    t1, t2 = _kth_threshold_keys2(keys, k1, k2)

    rowmax = jnp.max(attn, axis=1, keepdims=True)
    e = jnp.exp(attn - rowmax)
    m1 = (keys >= t1).astype(jnp.float32)
    m2 = (keys >= t2).astype(jnp.float32)
    s1 = jnp.sum(e * m1, axis=1, keepdims=True)
    s2 = jnp.sum(e * m2, axis=1, keepdims=True)
    wgt = e * (m1 * (a1_ref[...] / s1) + m2 * (a2_ref[...] / s2))

    out_h = jax.lax.dot_general(
        wgt, v_ref[0, 0], (((1,), (0,)), ((), ())),
        preferred_element_type=jnp.float32)
    contrib = jnp.dot(out_h, wp_ref[0],
                      preferred_element_type=jnp.float32)

    @pl.when(h == 0)
    def _():
        out_ref[0] = contrib + bp_ref[...]

    @pl.when(h != 0)
    def _():
        out_ref[0] = out_ref[0] + contrib


def kernel(x, y, Wq, Wkv, ln_g, ln_b, Wp, bp, a1, a2):
    num_heads = _HEADS
    B, C, H, W = x.shape
    hd = C // num_heads
    scale = hd ** (-0.5)
    N = H * W

    yy = _avg_pool(y, 3, 1) + _avg_pool(y, 5, 2) + _avg_pool(y, 7, 3)
    yy = yy.reshape(B, C, N).transpose(0, 2, 1)
    mu = yy.mean(-1, keepdims=True)
    var = ((yy - mu) ** 2).mean(-1, keepdims=True)
    yy = (yy - mu) / jnp.sqrt(var + 1e-5) * ln_g + ln_b
    xt = x.reshape(B, C, N).transpose(0, 2, 1)
    kv = (yy @ Wkv).reshape(B, N, 2, num_heads, hd).transpose(2, 0, 3, 1, 4)
    k_, v_ = kv[0], kv[1]
    q = (xt @ Wq).reshape(B, N, num_heads, hd).transpose(0, 2, 1, 3)

    wp3 = Wp.reshape(num_heads, hd, C)
    a1b = a1.reshape(1, 1)
    a2b = a2.reshape(1, 1)
    bp2 = bp.reshape(1, C)
    kk1, kk2 = N // 2, N // 3

    import functools
    body = functools.partial(_attn_body, np.float32(scale), kk1, kk2)

    xo = pl.pallas_call(
        body,
        grid=(B, num_heads),
        in_specs=[
            pl.BlockSpec((1, 1, N, hd), lambda b, h: (b, h, 0, 0)),
            pl.BlockSpec((1, 1, N, hd), lambda b, h: (b, h, 0, 0)),
            pl.BlockSpec((1, 1, N, hd), lambda b, h: (b, h, 0, 0)),
            pl.BlockSpec((1, hd, C), lambda b, h: (h, 0, 0)),
            pl.BlockSpec((1, 1), lambda b, h: (0, 0)),
            pl.BlockSpec((1, 1), lambda b, h: (0, 0)),
            pl.BlockSpec((1, C), lambda b, h: (0, 0)),
        ],
        out_specs=pl.BlockSpec((1, N, C), lambda b, h: (b, 0, 0)),
        out_shape=jax.ShapeDtypeStruct((B, N, C), jnp.float32),
    )(q, k_, v_, wp3, a1b, a2b, bp2)

    return xo.reshape(B, H, W, C).transpose(0, 3, 1, 2)
